# baseline (speedup 1.0000x reference)
import functools

import jax
import jax.numpy as jnp
from jax import lax
from jax.experimental import pallas as pl
from jax.experimental.pallas import tpu as pltpu
from jax.experimental.pallas import tpu_sc as plsc

N_NODES = 10000
N_PAD = 10240
NC, NS = 2, 16
CHUNK = 125
N_CHUNKS = 40
_H0 = 64
_H1 = CHUNK - _H0
BM = 400


def _dinv_block(d0_ref, d1_ref):
    return lax.rsqrt(d0_ref[...] + d1_ref[...] + 1.0)


def _mlp_body(x_ref, d0_ref, d1_ref, w0_ref, b0_ref, w1_ref, b1_ref, wc1_ref, z_ref):
    h = jnp.maximum(jnp.dot(x_ref[...], w0_ref[...],
                            preferred_element_type=jnp.float32) + b0_ref[...], 0.0)
    h = jnp.maximum(jnp.dot(h, w1_ref[...],
                            preferred_element_type=jnp.float32) + b1_ref[...], 0.0)
    g = jnp.dot(h, wc1_ref[...], preferred_element_type=jnp.float32)
    z = g * _dinv_block(d0_ref, d1_ref)
    z_ref[0] = z[:, :128]
    z_ref[1] = z[:, 128:]


def _mlp_stage(x, d0, d1, w0, b0, w1, b1, wc1):
    return pl.pallas_call(
        _mlp_body,
        grid=(N_NODES // BM,),
        in_specs=[
            pl.BlockSpec((BM, 256), lambda i: (i, 0)),
            pl.BlockSpec((BM, 1), lambda i: (i, 0)),
            pl.BlockSpec((BM, 1), lambda i: (i, 0)),
            pl.BlockSpec((256, 256), lambda i: (0, 0)),
            pl.BlockSpec((1, 256), lambda i: (0, 0)),
            pl.BlockSpec((256, 256), lambda i: (0, 0)),
            pl.BlockSpec((1, 256), lambda i: (0, 0)),
            pl.BlockSpec((256, 256), lambda i: (0, 0)),
        ],
        out_specs=pl.BlockSpec((2, BM, 128), lambda i: (0, i, 0)),
        out_shape=jax.ShapeDtypeStruct((2, N_PAD, 128), jnp.float32),
    )(x, d0, d1, w0, b0, w1, b1, wc1)


def _mid_body(y_ref, d0_ref, d1_ref, b1_ref, wc2_ref, z2_ref):
    dinv = _dinv_block(d0_ref, d1_ref)
    y = jnp.concatenate([y_ref[0], y_ref[1]], axis=1)
    h = jnp.maximum(y * dinv + b1_ref[...], 0.0)
    g = jnp.dot(h, wc2_ref[...], preferred_element_type=jnp.float32)
    z2_ref[...] = jnp.concatenate([g * dinv, jnp.zeros_like(g)], axis=1)


def _mid_stage(y, d0, d1, conv1_b, wc2):
    return pl.pallas_call(
        _mid_body,
        grid=(N_NODES // BM,),
        in_specs=[
            pl.BlockSpec((2, BM, 128), lambda i: (0, i, 0)),
            pl.BlockSpec((BM, 1), lambda i: (i, 0)),
            pl.BlockSpec((BM, 1), lambda i: (i, 0)),
            pl.BlockSpec((1, 256), lambda i: (0, 0)),
            pl.BlockSpec((256, 64), lambda i: (0, 0)),
        ],
        out_specs=pl.BlockSpec((BM, 128), lambda i: (i, 0)),
        out_shape=jax.ShapeDtypeStruct((N_PAD, 128), jnp.float32),
    )(y, d0, d1, conv1_b, wc2)


def _final_body(p_ref, z2_ref, d0_ref, d1_ref, b2_ref, out_ref):
    tot = p_ref[0] + p_ref[1] - z2_ref[...]
    out_ref[...] = tot[:, :64] * _dinv_block(d0_ref, d1_ref) + b2_ref[...]


def _final_stage(p, z2, d0, d1, conv2_b):
    return pl.pallas_call(
        _final_body,
        grid=(N_NODES // BM,),
        in_specs=[
            pl.BlockSpec((2, BM, 128), lambda i: (0, i, 0)),
            pl.BlockSpec((BM, 128), lambda i: (i, 0)),
            pl.BlockSpec((BM, 1), lambda i: (i, 0)),
            pl.BlockSpec((BM, 1), lambda i: (i, 0)),
            pl.BlockSpec((1, 64), lambda i: (0, 0)),
        ],
        out_specs=pl.BlockSpec((BM, 64), lambda i: (i, 0)),
        out_shape=jax.ShapeDtypeStruct((N_NODES, 64), jnp.float32),
    )(p, z2, d0, d1, conv2_b)


_ROWS = N_PAD // NS
_BODY_CHUNKS = 10


def _deg_stage(dst3):
    mesh = plsc.VectorSubcoreMesh(core_axis_name="c", subcore_axis_name="s")

    @functools.partial(
        pl.kernel,
        out_type=jax.ShapeDtypeStruct((NC * N_PAD,), jnp.float32),
        mesh=mesh,
        scratch_types=[
            pltpu.VMEM((N_CHUNKS, CHUNK), jnp.int32),
            pltpu.VMEM((CHUNK,), jnp.float32),
            pltpu.VMEM((_ROWS,), jnp.float32),
            pltpu.VMEM_SHARED((N_PAD,), jnp.float32),
        ],
    )
    def deg_kernel(dst_hbm, deg_out, idx_v, ones_v, zeros_v, acc):
        c = lax.axis_index("c")
        s = lax.axis_index("s")
        w = c * NS + s
        base = s * _ROWS
        for j in range(0, CHUNK - 15, 16):
            ones_v[pl.ds(j, 16)] = jnp.full((16,), 1.0, jnp.float32)
        ones_v[pl.ds(CHUNK - 16, 16)] = jnp.full((16,), 1.0, jnp.float32)
        for j in range(_ROWS // 16):
            zeros_v[pl.ds(j * 16, 16)] = jnp.zeros((16,), jnp.float32)
        pltpu.sync_copy(zeros_v, acc.at[pl.ds(base, _ROWS)])
        pltpu.sync_copy(dst_hbm.at[w], idx_v)
        plsc.subcore_barrier()

        @pl.loop(0, N_CHUNKS)
        def _(j):
            pltpu.sync_copy(ones_v, acc.at[idx_v.at[j]], add=True)

        plsc.subcore_barrier()
        pltpu.sync_copy(acc.at[pl.ds(base, _ROWS)],
                        deg_out.at[pl.ds(c * N_PAD + base, _ROWS)])

    return deg_kernel(dst3)


def _agg_stage(z, src3, dst3, d, feature_split):
    mesh = plsc.VectorSubcoreMesh(core_axis_name="c", subcore_axis_name="s")

    @functools.partial(
        pl.kernel,
        out_type=jax.ShapeDtypeStruct((NC * N_PAD, d), jnp.float32),
        mesh=mesh,
        scratch_types=[
            pltpu.VMEM((N_CHUNKS, CHUNK), jnp.int32),
            pltpu.VMEM((N_CHUNKS, CHUNK), jnp.int32),
            pltpu.VMEM((CHUNK, d), jnp.float32),
            pltpu.VMEM((CHUNK, d), jnp.float32),
            pltpu.VMEM_SHARED((N_PAD, d), jnp.float32),
            pltpu.SemaphoreType.DMA,
            pltpu.SemaphoreType.DMA,
            pltpu.SemaphoreType.DMA,
            pltpu.SemaphoreType.DMA,
        ],
    )
    def agg_kernel(z_hbm, src_hbm, dst_hbm, y_out,
                   src_v, dst_v, buf_a, buf_b, acc,
                   sem_a, sem_b, sem_c, sem_d):
        c = lax.axis_index("c")
        s = lax.axis_index("s")
        base = s * _ROWS
        table = z_hbm.at[c] if feature_split else z_hbm
        pltpu.sync_copy(table.at[pl.ds(base, _ROWS)], acc.at[pl.ds(base, _ROWS)])
        plsc.subcore_barrier()

        bufs = (buf_a, buf_b)
        sems = ((sem_a, sem_b), (sem_c, sem_d))

        def gather2(j, b):
            idx = src_v.at[j]
            d0 = pltpu.async_copy(table.at[idx.at[pl.ds(0, _H0)]],
                                  bufs[b].at[pl.ds(0, _H0)], sems[b][0])
            d1 = pltpu.async_copy(table.at[idx.at[pl.ds(_H0, _H1)]],
                                  bufs[b].at[pl.ds(_H0, _H1)], sems[b][1])
            return (d0, d1)

        workers = [2 * s, 2 * s + 1] if feature_split else [c * NS + s]
        for w in workers:
            pltpu.sync_copy(src_hbm.at[w], src_v)
            pltpu.sync_copy(dst_hbm.at[w], dst_v)

            @pl.loop(0, N_CHUNKS // _BODY_CHUNKS)
            def _(i):
                j0 = i * _BODY_CHUNKS
                pend = [None] * _BODY_CHUNKS
                pend[0] = gather2(j0, 0)
                for k in range(_BODY_CHUNKS):
                    if k + 1 < _BODY_CHUNKS:
                        pend[k + 1] = gather2(j0 + k + 1, (k + 1) % 2)
                    pend[k][0].wait()
                    pend[k][1].wait()
                    pltpu.sync_copy(bufs[k % 2],
                                    acc.at[dst_v.at[j0 + k]], add=True)

        plsc.subcore_barrier()
        pltpu.sync_copy(acc.at[pl.ds(base, _ROWS)],
                        y_out.at[pl.ds(c * N_PAD + base, _ROWS)])

    return agg_kernel(z, src3, dst3)


def kernel(x, edge_index, L0_W, L0_b, L1_W, L1_b, conv1_W, conv1_b, conv2_W, conv2_b):
    n = x.shape[0]
    del n

    src3 = edge_index[0].astype(jnp.int32).reshape(NC * NS, N_CHUNKS, CHUNK)
    dst3 = edge_index[1].astype(jnp.int32).reshape(NC * NS, N_CHUNKS, CHUNK)

    deg_flat = _deg_stage(dst3)
    d0 = deg_flat[:N_PAD, None]
    d1 = deg_flat[N_PAD:, None]

    b0 = L0_b.reshape(1, -1)
    b1 = L1_b.reshape(1, -1)
    bc1 = conv1_b.reshape(1, -1)
    bc2 = conv2_b.reshape(1, -1)

    z = _mlp_stage(x, d0, d1, L0_W, b0, L1_W, b1, conv1_W)

    y = _agg_stage(z, src3, dst3, 128, feature_split=True).reshape(NC, N_PAD, 128)

    z2 = _mid_stage(y, d0, d1, bc1, conv2_W)

    p = _agg_stage(z2, src3, dst3, 128, feature_split=False).reshape(NC, N_PAD, 128)

    return _final_stage(p, z2, d0, d1, bc2)


# speedup vs baseline: 1.0372x; 1.0372x over previous
import functools

import jax
import jax.numpy as jnp
from jax import lax
from jax.experimental import pallas as pl
from jax.experimental.pallas import tpu as pltpu
from jax.experimental.pallas import tpu_sc as plsc

N_NODES = 10000
N_PAD = 10240
NC, NS = 2, 16
CHUNK = 125
N_CHUNKS = 40
BM = 400


def _dinv_block(d0_ref, d1_ref):
    return lax.rsqrt(d0_ref[...] + d1_ref[...] + 1.0)


def _mlp_body(x_ref, d0_ref, d1_ref, w0_ref, b0_ref, w1_ref, b1_ref, wc1_ref, z_ref):
    h = jnp.maximum(jnp.dot(x_ref[...], w0_ref[...],
                            preferred_element_type=jnp.float32) + b0_ref[...], 0.0)
    h = jnp.maximum(jnp.dot(h, w1_ref[...],
                            preferred_element_type=jnp.float32) + b1_ref[...], 0.0)
    g = jnp.dot(h, wc1_ref[...], preferred_element_type=jnp.float32)
    z = g * _dinv_block(d0_ref, d1_ref)
    z_ref[0] = z[:, :128]
    z_ref[1] = z[:, 128:]


def _mlp_stage(x, d0, d1, w0, b0, w1, b1, wc1):
    return pl.pallas_call(
        _mlp_body,
        grid=(N_NODES // BM,),
        in_specs=[
            pl.BlockSpec((BM, 256), lambda i: (i, 0)),
            pl.BlockSpec((BM, 1), lambda i: (i, 0)),
            pl.BlockSpec((BM, 1), lambda i: (i, 0)),
            pl.BlockSpec((256, 256), lambda i: (0, 0)),
            pl.BlockSpec((1, 256), lambda i: (0, 0)),
            pl.BlockSpec((256, 256), lambda i: (0, 0)),
            pl.BlockSpec((1, 256), lambda i: (0, 0)),
            pl.BlockSpec((256, 256), lambda i: (0, 0)),
        ],
        out_specs=pl.BlockSpec((2, BM, 128), lambda i: (0, i, 0)),
        out_shape=jax.ShapeDtypeStruct((2, N_PAD, 128), jnp.float32),
    )(x, d0, d1, w0, b0, w1, b1, wc1)


def _mid_body(y_ref, d0_ref, d1_ref, b1_ref, wc2_ref, z2_ref):
    dinv = _dinv_block(d0_ref, d1_ref)
    y = jnp.concatenate([y_ref[0], y_ref[1]], axis=1)
    h = jnp.maximum(y * dinv + b1_ref[...], 0.0)
    g = jnp.dot(h, wc2_ref[...], preferred_element_type=jnp.float32)
    z2_ref[...] = jnp.concatenate([g * dinv, jnp.zeros_like(g)], axis=1)


def _mid_stage(y, d0, d1, conv1_b, wc2):
    return pl.pallas_call(
        _mid_body,
        grid=(N_NODES // BM,),
        in_specs=[
            pl.BlockSpec((2, BM, 128), lambda i: (0, i, 0)),
            pl.BlockSpec((BM, 1), lambda i: (i, 0)),
            pl.BlockSpec((BM, 1), lambda i: (i, 0)),
            pl.BlockSpec((1, 256), lambda i: (0, 0)),
            pl.BlockSpec((256, 64), lambda i: (0, 0)),
        ],
        out_specs=pl.BlockSpec((BM, 128), lambda i: (i, 0)),
        out_shape=jax.ShapeDtypeStruct((N_PAD, 128), jnp.float32),
    )(y, d0, d1, conv1_b, wc2)


def _final_body(p_ref, z2_ref, d0_ref, d1_ref, b2_ref, out_ref):
    tot = p_ref[0] + p_ref[1] - z2_ref[...]
    out_ref[...] = tot[:, :64] * _dinv_block(d0_ref, d1_ref) + b2_ref[...]


def _final_stage(p, z2, d0, d1, conv2_b):
    return pl.pallas_call(
        _final_body,
        grid=(N_NODES // BM,),
        in_specs=[
            pl.BlockSpec((2, BM, 128), lambda i: (0, i, 0)),
            pl.BlockSpec((BM, 128), lambda i: (i, 0)),
            pl.BlockSpec((BM, 1), lambda i: (i, 0)),
            pl.BlockSpec((BM, 1), lambda i: (i, 0)),
            pl.BlockSpec((1, 64), lambda i: (0, 0)),
        ],
        out_specs=pl.BlockSpec((BM, 64), lambda i: (i, 0)),
        out_shape=jax.ShapeDtypeStruct((N_NODES, 64), jnp.float32),
    )(p, z2, d0, d1, conv2_b)


_ROWS = N_PAD // NS
_BODY_CHUNKS = 20


def _deg_stage(dst3):
    mesh = plsc.VectorSubcoreMesh(core_axis_name="c", subcore_axis_name="s")

    @functools.partial(
        pl.kernel,
        out_type=jax.ShapeDtypeStruct((NC * N_PAD,), jnp.float32),
        mesh=mesh,
        scratch_types=[
            pltpu.VMEM((N_CHUNKS, CHUNK), jnp.int32),
            pltpu.VMEM((CHUNK,), jnp.float32),
            pltpu.VMEM((_ROWS,), jnp.float32),
            pltpu.VMEM_SHARED((N_PAD,), jnp.float32),
        ],
    )
    def deg_kernel(dst_hbm, deg_out, idx_v, ones_v, zeros_v, acc):
        c = lax.axis_index("c")
        s = lax.axis_index("s")
        w = c * NS + s
        base = s * _ROWS
        for j in range(0, CHUNK - 15, 16):
            ones_v[pl.ds(j, 16)] = jnp.full((16,), 1.0, jnp.float32)
        ones_v[pl.ds(CHUNK - 16, 16)] = jnp.full((16,), 1.0, jnp.float32)
        for j in range(_ROWS // 16):
            zeros_v[pl.ds(j * 16, 16)] = jnp.zeros((16,), jnp.float32)
        pltpu.sync_copy(zeros_v, acc.at[pl.ds(base, _ROWS)])
        pltpu.sync_copy(dst_hbm.at[w], idx_v)
        plsc.subcore_barrier()

        @pl.loop(0, N_CHUNKS)
        def _(j):
            pltpu.sync_copy(ones_v, acc.at[idx_v.at[j]], add=True)

        plsc.subcore_barrier()
        pltpu.sync_copy(acc.at[pl.ds(base, _ROWS)],
                        deg_out.at[pl.ds(c * N_PAD + base, _ROWS)])

    return deg_kernel(dst3)


def _agg_stage(z, src3, dst3, d, feature_split):
    mesh = plsc.VectorSubcoreMesh(core_axis_name="c", subcore_axis_name="s")

    @functools.partial(
        pl.kernel,
        out_type=jax.ShapeDtypeStruct((NC * N_PAD, d), jnp.float32),
        mesh=mesh,
        scratch_types=[
            pltpu.VMEM((N_CHUNKS, CHUNK), jnp.int32),
            pltpu.VMEM((N_CHUNKS, CHUNK), jnp.int32),
            pltpu.VMEM((CHUNK, d), jnp.float32),
            pltpu.VMEM((CHUNK, d), jnp.float32),
            pltpu.VMEM_SHARED((N_PAD, d), jnp.float32),
            pltpu.SemaphoreType.DMA,
            pltpu.SemaphoreType.DMA,
        ],
    )
    def agg_kernel(z_hbm, src_hbm, dst_hbm, y_out,
                   src_v, dst_v, buf_a, buf_b, acc, sem_a, sem_b):
        c = lax.axis_index("c")
        s = lax.axis_index("s")
        base = s * _ROWS
        table = z_hbm.at[c] if feature_split else z_hbm
        pltpu.sync_copy(table.at[pl.ds(base, _ROWS)], acc.at[pl.ds(base, _ROWS)])
        plsc.subcore_barrier()

        bufs = (buf_a, buf_b)
        sems = (sem_a, sem_b)

        workers = [2 * s, 2 * s + 1] if feature_split else [c * NS + s]
        for w in workers:
            pltpu.sync_copy(src_hbm.at[w], src_v)
            pltpu.sync_copy(dst_hbm.at[w], dst_v)

            @pl.loop(0, N_CHUNKS // _BODY_CHUNKS)
            def _(i):
                j0 = i * _BODY_CHUNKS
                pend = [None] * _BODY_CHUNKS
                pend[0] = pltpu.async_copy(
                    table.at[src_v.at[j0]], bufs[0], sems[0])
                for k in range(_BODY_CHUNKS):
                    if k + 1 < _BODY_CHUNKS:
                        pend[k + 1] = pltpu.async_copy(
                            table.at[src_v.at[j0 + k + 1]], bufs[(k + 1) % 2],
                            sems[(k + 1) % 2])
                    pend[k].wait()
                    pltpu.sync_copy(bufs[k % 2],
                                    acc.at[dst_v.at[j0 + k]], add=True)

        plsc.subcore_barrier()
        pltpu.sync_copy(acc.at[pl.ds(base, _ROWS)],
                        y_out.at[pl.ds(c * N_PAD + base, _ROWS)])

    return agg_kernel(z, src3, dst3)


def kernel(x, edge_index, L0_W, L0_b, L1_W, L1_b, conv1_W, conv1_b, conv2_W, conv2_b):
    n = x.shape[0]
    del n

    src3 = edge_index[0].astype(jnp.int32).reshape(NC * NS, N_CHUNKS, CHUNK)
    dst3 = edge_index[1].astype(jnp.int32).reshape(NC * NS, N_CHUNKS, CHUNK)

    deg_flat = _deg_stage(dst3)
    d0 = deg_flat[:N_PAD, None]
    d1 = deg_flat[N_PAD:, None]

    b0 = L0_b.reshape(1, -1)
    b1 = L1_b.reshape(1, -1)
    bc1 = conv1_b.reshape(1, -1)
    bc2 = conv2_b.reshape(1, -1)

    z = _mlp_stage(x, d0, d1, L0_W, b0, L1_W, b1, conv1_W)

    y = _agg_stage(z, src3, dst3, 128, feature_split=True).reshape(NC, N_PAD, 128)

    z2 = _mid_stage(y, d0, d1, bc1, conv2_W)

    p = _agg_stage(z2, src3, dst3, 128, feature_split=False).reshape(NC, N_PAD, 128)

    return _final_stage(p, z2, d0, d1, bc2)


# speedup vs baseline: 1.0424x; 1.0049x over previous
import functools

import jax
import jax.numpy as jnp
from jax import lax
from jax.experimental import pallas as pl
from jax.experimental.pallas import tpu as pltpu
from jax.experimental.pallas import tpu_sc as plsc

N_NODES = 10000
N_PAD = 10240
NC, NS = 2, 16
CHUNK = 125
N_CHUNKS = 40
BM = 400


def _dinv_block(d0_ref, d1_ref):
    return lax.rsqrt(d0_ref[...] + d1_ref[...] + 1.0)


def _mlp_body(x_ref, d0_ref, d1_ref, w0_ref, b0_ref, w1_ref, b1_ref, wc1_ref, z_ref):
    h = jnp.maximum(jnp.dot(x_ref[...], w0_ref[...],
                            preferred_element_type=jnp.float32) + b0_ref[...], 0.0)
    h = jnp.maximum(jnp.dot(h, w1_ref[...],
                            preferred_element_type=jnp.float32) + b1_ref[...], 0.0)
    g = jnp.dot(h, wc1_ref[...], preferred_element_type=jnp.float32)
    z = g * _dinv_block(d0_ref, d1_ref)
    z_ref[0] = z[:, :128]
    z_ref[1] = z[:, 128:]


def _mlp_stage(x, d0, d1, w0, b0, w1, b1, wc1):
    return pl.pallas_call(
        _mlp_body,
        grid=(N_NODES // BM,),
        in_specs=[
            pl.BlockSpec((BM, 256), lambda i: (i, 0)),
            pl.BlockSpec((BM, 1), lambda i: (i, 0)),
            pl.BlockSpec((BM, 1), lambda i: (i, 0)),
            pl.BlockSpec((256, 256), lambda i: (0, 0)),
            pl.BlockSpec((1, 256), lambda i: (0, 0)),
            pl.BlockSpec((256, 256), lambda i: (0, 0)),
            pl.BlockSpec((1, 256), lambda i: (0, 0)),
            pl.BlockSpec((256, 256), lambda i: (0, 0)),
        ],
        out_specs=pl.BlockSpec((2, BM, 128), lambda i: (0, i, 0)),
        out_shape=jax.ShapeDtypeStruct((2, N_PAD, 128), jnp.float32),
    )(x, d0, d1, w0, b0, w1, b1, wc1)


def _mid_body(y_ref, d0_ref, d1_ref, b1_ref, wc2_ref, z2_ref):
    dinv = _dinv_block(d0_ref, d1_ref)
    y = jnp.concatenate([y_ref[0], y_ref[1]], axis=1)
    h = jnp.maximum(y * dinv + b1_ref[...], 0.0)
    g = jnp.dot(h, wc2_ref[...], preferred_element_type=jnp.float32)
    z2_ref[...] = jnp.concatenate([g * dinv, jnp.zeros_like(g)], axis=1)


def _mid_stage(y, d0, d1, conv1_b, wc2):
    return pl.pallas_call(
        _mid_body,
        grid=(N_NODES // BM,),
        in_specs=[
            pl.BlockSpec((2, BM, 128), lambda i: (0, i, 0)),
            pl.BlockSpec((BM, 1), lambda i: (i, 0)),
            pl.BlockSpec((BM, 1), lambda i: (i, 0)),
            pl.BlockSpec((1, 256), lambda i: (0, 0)),
            pl.BlockSpec((256, 64), lambda i: (0, 0)),
        ],
        out_specs=pl.BlockSpec((BM, 128), lambda i: (i, 0)),
        out_shape=jax.ShapeDtypeStruct((N_PAD, 128), jnp.float32),
    )(y, d0, d1, conv1_b, wc2)


def _final_body(p_ref, z2_ref, d0_ref, d1_ref, b2_ref, out_ref):
    tot = p_ref[0] + p_ref[1] - z2_ref[...]
    out_ref[...] = tot[:, :64] * _dinv_block(d0_ref, d1_ref) + b2_ref[...]


def _final_stage(p, z2, d0, d1, conv2_b):
    return pl.pallas_call(
        _final_body,
        grid=(N_NODES // BM,),
        in_specs=[
            pl.BlockSpec((2, BM, 128), lambda i: (0, i, 0)),
            pl.BlockSpec((BM, 128), lambda i: (i, 0)),
            pl.BlockSpec((BM, 1), lambda i: (i, 0)),
            pl.BlockSpec((BM, 1), lambda i: (i, 0)),
            pl.BlockSpec((1, 64), lambda i: (0, 0)),
        ],
        out_specs=pl.BlockSpec((BM, 64), lambda i: (i, 0)),
        out_shape=jax.ShapeDtypeStruct((N_NODES, 64), jnp.float32),
    )(p, z2, d0, d1, conv2_b)


_ROWS = N_PAD // NS
_BODY_CHUNKS = 40


def _deg_stage(dst3):
    mesh = plsc.VectorSubcoreMesh(core_axis_name="c", subcore_axis_name="s")

    @functools.partial(
        pl.kernel,
        out_type=jax.ShapeDtypeStruct((NC * N_PAD,), jnp.float32),
        mesh=mesh,
        scratch_types=[
            pltpu.VMEM((N_CHUNKS, CHUNK), jnp.int32),
            pltpu.VMEM((CHUNK,), jnp.float32),
            pltpu.VMEM((_ROWS,), jnp.float32),
            pltpu.VMEM_SHARED((N_PAD,), jnp.float32),
        ],
    )
    def deg_kernel(dst_hbm, deg_out, idx_v, ones_v, zeros_v, acc):
        c = lax.axis_index("c")
        s = lax.axis_index("s")
        w = c * NS + s
        base = s * _ROWS
        for j in range(0, CHUNK - 15, 16):
            ones_v[pl.ds(j, 16)] = jnp.full((16,), 1.0, jnp.float32)
        ones_v[pl.ds(CHUNK - 16, 16)] = jnp.full((16,), 1.0, jnp.float32)
        for j in range(_ROWS // 16):
            zeros_v[pl.ds(j * 16, 16)] = jnp.zeros((16,), jnp.float32)
        pltpu.sync_copy(zeros_v, acc.at[pl.ds(base, _ROWS)])
        pltpu.sync_copy(dst_hbm.at[w], idx_v)
        plsc.subcore_barrier()

        @pl.loop(0, N_CHUNKS)
        def _(j):
            pltpu.sync_copy(ones_v, acc.at[idx_v.at[j]], add=True)

        plsc.subcore_barrier()
        pltpu.sync_copy(acc.at[pl.ds(base, _ROWS)],
                        deg_out.at[pl.ds(c * N_PAD + base, _ROWS)])

    return deg_kernel(dst3)


def _agg_stage(z, src3, dst3, d, feature_split):
    mesh = plsc.VectorSubcoreMesh(core_axis_name="c", subcore_axis_name="s")

    @functools.partial(
        pl.kernel,
        out_type=jax.ShapeDtypeStruct((NC * N_PAD, d), jnp.float32),
        mesh=mesh,
        scratch_types=[
            pltpu.VMEM((N_CHUNKS, CHUNK), jnp.int32),
            pltpu.VMEM((N_CHUNKS, CHUNK), jnp.int32),
            pltpu.VMEM((CHUNK, d), jnp.float32),
            pltpu.VMEM((CHUNK, d), jnp.float32),
            pltpu.VMEM_SHARED((N_PAD, d), jnp.float32),
            pltpu.SemaphoreType.DMA,
            pltpu.SemaphoreType.DMA,
        ],
    )
    def agg_kernel(z_hbm, src_hbm, dst_hbm, y_out,
                   src_v, dst_v, buf_a, buf_b, acc, sem_a, sem_b):
        c = lax.axis_index("c")
        s = lax.axis_index("s")
        base = s * _ROWS
        table = z_hbm.at[c] if feature_split else z_hbm
        pltpu.sync_copy(table.at[pl.ds(base, _ROWS)], acc.at[pl.ds(base, _ROWS)])
        plsc.subcore_barrier()

        bufs = (buf_a, buf_b)
        sems = (sem_a, sem_b)

        workers = [2 * s, 2 * s + 1] if feature_split else [c * NS + s]
        for w in workers:
            pltpu.sync_copy(src_hbm.at[w], src_v)
            pltpu.sync_copy(dst_hbm.at[w], dst_v)

            @pl.loop(0, N_CHUNKS // _BODY_CHUNKS)
            def _(i):
                j0 = i * _BODY_CHUNKS
                pend = [None] * _BODY_CHUNKS
                pend[0] = pltpu.async_copy(
                    table.at[src_v.at[j0]], bufs[0], sems[0])
                for k in range(_BODY_CHUNKS):
                    if k + 1 < _BODY_CHUNKS:
                        pend[k + 1] = pltpu.async_copy(
                            table.at[src_v.at[j0 + k + 1]], bufs[(k + 1) % 2],
                            sems[(k + 1) % 2])
                    pend[k].wait()
                    pltpu.sync_copy(bufs[k % 2],
                                    acc.at[dst_v.at[j0 + k]], add=True)

        plsc.subcore_barrier()
        pltpu.sync_copy(acc.at[pl.ds(base, _ROWS)],
                        y_out.at[pl.ds(c * N_PAD + base, _ROWS)])

    return agg_kernel(z, src3, dst3)


def kernel(x, edge_index, L0_W, L0_b, L1_W, L1_b, conv1_W, conv1_b, conv2_W, conv2_b):
    n = x.shape[0]
    del n

    src3 = edge_index[0].astype(jnp.int32).reshape(NC * NS, N_CHUNKS, CHUNK)
    dst3 = edge_index[1].astype(jnp.int32).reshape(NC * NS, N_CHUNKS, CHUNK)

    deg_flat = _deg_stage(dst3)
    d0 = deg_flat[:N_PAD, None]
    d1 = deg_flat[N_PAD:, None]

    b0 = L0_b.reshape(1, -1)
    b1 = L1_b.reshape(1, -1)
    bc1 = conv1_b.reshape(1, -1)
    bc2 = conv2_b.reshape(1, -1)

    z = _mlp_stage(x, d0, d1, L0_W, b0, L1_W, b1, conv1_W)

    y = _agg_stage(z, src3, dst3, 128, feature_split=True).reshape(NC, N_PAD, 128)

    z2 = _mid_stage(y, d0, d1, bc1, conv2_W)

    p = _agg_stage(z2, src3, dst3, 128, feature_split=False).reshape(NC, N_PAD, 128)

    return _final_stage(p, z2, d0, d1, bc2)
